# Initial kernel scaffold; baseline (speedup 1.0000x reference)
import jax
import jax.numpy as jnp
from jax.experimental import pallas as pl


def kernel(x, edge_index, edge_attr, W1, b1, W2, b2, W3, b3, Wl1, bl1, Wl2, bl2):
    raise NotImplementedError("write your pallas kernel here")


# speedup vs baseline: 16.1537x; 16.1537x over previous
import functools

import jax
import jax.numpy as jnp
from jax import lax
from jax.experimental import pallas as pl
from jax.experimental.pallas import tpu as pltpu
from jax.experimental.pallas import tpu_sc as plsc

_N = 10000
_E = 320000
_CH = 128
_CPT = 80
_EPAD = 32 * _CPT * _CH
_ZR = 2000

_mesh = plsc.VectorSubcoreMesh(core_axis_name="c", subcore_axis_name="s",
                               num_cores=2, num_subcores=16)


@functools.partial(
    pl.kernel,
    out_type=[jax.ShapeDtypeStruct((_N,), jnp.float32),
              jax.ShapeDtypeStruct((_N,), jnp.float32)],
    mesh=_mesh,
    scratch_types=[
        pltpu.VMEM((_CPT, _CH), jnp.int32),
        pltpu.VMEM((_CPT, _CH), jnp.float32),
        pltpu.VMEM_SHARED((_N,), jnp.float32),
    ],
)
def _sc_deg(dst_h, ew_h, z_h, out0_h, out1_h, dstb, ewb, acc):
    c = lax.axis_index("c")
    s = lax.axis_index("s")
    row0 = c * (16 * _CPT) + s * _CPT
    pltpu.sync_copy(dst_h.at[pl.ds(row0, _CPT), :], dstb)
    pltpu.sync_copy(ew_h.at[pl.ds(row0, _CPT), :], ewb)

    @pl.when(s == 0)
    def _zero():
        pltpu.sync_copy(z_h, acc)

    plsc.subcore_barrier()

    def chunk(i, carry):
        pltpu.sync_copy(ewb.at[i], acc.at[dstb.at[i]], add=True)
        return carry

    lax.fori_loop(0, _CPT, chunk, 0)
    plsc.subcore_barrier()

    @pl.when((s == 0) & (c == 0))
    def _drain0():
        pltpu.sync_copy(acc, out0_h)

    @pl.when((s == 0) & (c == 1))
    def _drain1():
        pltpu.sync_copy(acc, out1_h)


def _make_prop(F):
    @functools.partial(
        pl.kernel,
        out_type=jax.ShapeDtypeStruct((2, _N, F), jnp.float32),
        mesh=_mesh,
        scratch_types=[
            pltpu.VMEM((_CPT, _CH), jnp.int32),
            pltpu.VMEM((_CPT, _CH), jnp.int32),
            pltpu.VMEM((_CPT, _CH), jnp.float32),
            pltpu.VMEM((_CH, F), jnp.float32),
            pltpu.VMEM_SHARED((_N, F), jnp.float32),
            pltpu.SemaphoreType.DMA,
        ],
    )
    def prop(src_h, dst_h, ew_h, u_h, z_h, out_h, srcb, dstb, ewb, rows, acc,
             sem):
        c = lax.axis_index("c")
        s = lax.axis_index("s")
        row0 = c * (16 * _CPT) + s * _CPT
        pltpu.sync_copy(src_h.at[pl.ds(row0, _CPT), :], srcb)
        pltpu.sync_copy(dst_h.at[pl.ds(row0, _CPT), :], dstb)
        pltpu.sync_copy(ew_h.at[pl.ds(row0, _CPT), :], ewb)

        @pl.when(s < 5)
        def _zero():
            pltpu.sync_copy(z_h, acc.at[pl.ds(s * _ZR, _ZR), :])

        plsc.subcore_barrier()

        def chunk(i, carry):
            pltpu.async_copy(u_h.at[srcb.at[i]], rows, sem).wait()

            def scale(j, c2):
                wv = ewb[i, pl.ds(j * 16, 16)]
                for l in range(16):
                    e = j * 16 + l
                    w = wv[l]
                    for f in range(F // 16):
                        sl = pl.ds(f * 16, 16)
                        rows[e, sl] = rows[e, sl] * w
                return c2

            lax.fori_loop(0, _CH // 16, scale, 0)
            pltpu.sync_copy(rows, acc.at[dstb.at[i]], add=True)
            return carry

        lax.fori_loop(0, _CPT, chunk, 0)
        plsc.subcore_barrier()

        @pl.when(s < 5)
        def _drain():
            pltpu.sync_copy(acc.at[pl.ds(s * _ZR, _ZR), :],
                            out_h.at[c, pl.ds(s * _ZR, _ZR), :])

    return prop


_prop128 = _make_prop(128)

_R = 2000


def _tc_prep(deg_t, x, W1):
    def body(deg_ref, x_ref, w_ref, h_ref, u_ref, dv_ref, dv2_ref):
        dsum = deg_ref[:, 0:1] + deg_ref[:, 1:2] + 1.0
        dv = lax.rsqrt(dsum)
        h = jnp.dot(x_ref[...], w_ref[...], preferred_element_type=jnp.float32)
        h_ref[...] = h
        u_ref[...] = jnp.concatenate([h * dv, jnp.zeros_like(h)], axis=1)
        dv_ref[...] = dv
        dv2_ref[...] = dv * dv

    return pl.pallas_call(
        body,
        grid=(_N // _R,),
        in_specs=[
            pl.BlockSpec((_R, 2), lambda r: (r, 0)),
            pl.BlockSpec((_R, 128), lambda r: (r, 0)),
            pl.BlockSpec((128, 64), lambda r: (0, 0)),
        ],
        out_specs=[
            pl.BlockSpec((_R, 64), lambda r: (r, 0)),
            pl.BlockSpec((_R, 128), lambda r: (r, 0)),
            pl.BlockSpec((_R, 1), lambda r: (r, 0)),
            pl.BlockSpec((_R, 1), lambda r: (r, 0)),
        ],
        out_shape=[
            jax.ShapeDtypeStruct((_N, 64), jnp.float32),
            jax.ShapeDtypeStruct((_N, 128), jnp.float32),
            jax.ShapeDtypeStruct((_N, 1), jnp.float32),
            jax.ShapeDtypeStruct((_N, 1), jnp.float32),
        ],
    )(deg_t, x, W1)


def _tc_l1(sa, sb, h1, dv, dv2, b1, W2):
    def body(sa_ref, sb_ref, h_ref, dv_ref, dv2_ref, b_ref, w_ref,
             y_ref, u_ref):
        t = (dv_ref[...] * (sa_ref[...] + sb_ref[...])[:, 0:64]
             + dv2_ref[...] * h_ref[...] + b_ref[...])
        z1 = jnp.maximum(t, 0.0)
        y2 = jnp.dot(z1, w_ref[...], preferred_element_type=jnp.float32)
        y_ref[...] = y2
        u_ref[...] = dv_ref[...] * y2

    return pl.pallas_call(
        body,
        grid=(_N // _R,),
        in_specs=[
            pl.BlockSpec((_R, 128), lambda r: (r, 0)),
            pl.BlockSpec((_R, 128), lambda r: (r, 0)),
            pl.BlockSpec((_R, 64), lambda r: (r, 0)),
            pl.BlockSpec((_R, 1), lambda r: (r, 0)),
            pl.BlockSpec((_R, 1), lambda r: (r, 0)),
            pl.BlockSpec((1, 64), lambda r: (0, 0)),
            pl.BlockSpec((64, 128), lambda r: (0, 0)),
        ],
        out_specs=[
            pl.BlockSpec((_R, 128), lambda r: (r, 0)),
            pl.BlockSpec((_R, 128), lambda r: (r, 0)),
        ],
        out_shape=[
            jax.ShapeDtypeStruct((_N, 128), jnp.float32),
            jax.ShapeDtypeStruct((_N, 128), jnp.float32),
        ],
    )(sa, sb, h1, dv, dv2, b1, W2)


def _tc_l2(sa, sb, y2, dv, dv2, b2):
    def body(sa_ref, sb_ref, y_ref, dv_ref, dv2_ref, b_ref, z_ref, u_ref):
        t = (dv_ref[...] * (sa_ref[...] + sb_ref[...])
             + dv2_ref[...] * y_ref[...] + b_ref[...])
        z2 = jnp.maximum(t, 0.0)
        z_ref[...] = z2
        u_ref[...] = dv_ref[...] * z2

    return pl.pallas_call(
        body,
        grid=(_N // _R,),
        in_specs=[
            pl.BlockSpec((_R, 128), lambda r: (r, 0)),
            pl.BlockSpec((_R, 128), lambda r: (r, 0)),
            pl.BlockSpec((_R, 128), lambda r: (r, 0)),
            pl.BlockSpec((_R, 1), lambda r: (r, 0)),
            pl.BlockSpec((_R, 1), lambda r: (r, 0)),
            pl.BlockSpec((1, 128), lambda r: (0, 0)),
        ],
        out_specs=[
            pl.BlockSpec((_R, 128), lambda r: (r, 0)),
            pl.BlockSpec((_R, 128), lambda r: (r, 0)),
        ],
        out_shape=[
            jax.ShapeDtypeStruct((_N, 128), jnp.float32),
            jax.ShapeDtypeStruct((_N, 128), jnp.float32),
        ],
    )(sa, sb, y2, dv, dv2, b2)


def _tc_final(sa, sb, z2, dv, dv2, W3, b3, Wl1, bl1, Wl2, bl2):
    G = _N // _R

    def body(sa_ref, sb_ref, z2_ref, dv_ref, dv2_ref, w3_ref, b3_ref,
             wl1_ref, bl1_ref, wl2_ref, bl2_ref, out_ref, gmax):
        r = pl.program_id(0)
        t = (dv_ref[...] * (sa_ref[...] + sb_ref[...])
             + dv2_ref[...] * z2_ref[...])
        h3 = jnp.maximum(
            jnp.dot(t, w3_ref[...], preferred_element_type=jnp.float32)
            + b3_ref[...], 0.0)
        m = jnp.broadcast_to(jnp.max(h3, axis=0, keepdims=True), (8, 256))

        @pl.when(r == 0)
        def _init():
            gmax[...] = m

        @pl.when(r > 0)
        def _acc():
            gmax[...] = jnp.maximum(gmax[...], m)

        @pl.when(r == G - 1)
        def _fin():
            g = gmax[0:1, :]
            a = jnp.maximum(
                jnp.dot(g, wl1_ref[...], preferred_element_type=jnp.float32)
                + bl1_ref[...], 0.0)
            logits = (jnp.dot(a, wl2_ref[...],
                              preferred_element_type=jnp.float32)
                      + bl2_ref[...])
            m0 = jnp.max(logits, axis=1, keepdims=True)
            lse = m0 + jnp.log(
                jnp.sum(jnp.exp(logits - m0), axis=1, keepdims=True))
            out_ref[...] = logits - lse

    return pl.pallas_call(
        body,
        grid=(G,),
        in_specs=[
            pl.BlockSpec((_R, 128), lambda r: (r, 0)),
            pl.BlockSpec((_R, 128), lambda r: (r, 0)),
            pl.BlockSpec((_R, 128), lambda r: (r, 0)),
            pl.BlockSpec((_R, 1), lambda r: (r, 0)),
            pl.BlockSpec((_R, 1), lambda r: (r, 0)),
            pl.BlockSpec((128, 256), lambda r: (0, 0)),
            pl.BlockSpec((1, 256), lambda r: (0, 0)),
            pl.BlockSpec((256, 128), lambda r: (0, 0)),
            pl.BlockSpec((1, 128), lambda r: (0, 0)),
            pl.BlockSpec((128, 10), lambda r: (0, 0)),
            pl.BlockSpec((1, 10), lambda r: (0, 0)),
        ],
        out_specs=pl.BlockSpec((1, 10), lambda r: (0, 0)),
        out_shape=jax.ShapeDtypeStruct((1, 10), jnp.float32),
        scratch_shapes=[pltpu.VMEM((8, 256), jnp.float32)],
    )(sa, sb, z2, dv, dv2, W3, b3, Wl1, bl1, Wl2, bl2)


def kernel(x, edge_index, edge_attr, W1, b1, W2, b2, W3, b3, Wl1, bl1, Wl2,
           bl2):
    pad = _EPAD - _E
    fill = jnp.arange(pad, dtype=jnp.int32)
    src2d = jnp.concatenate([edge_index[0], fill]).reshape(_EPAD // _CH, _CH)
    dst2d = jnp.concatenate([edge_index[1], fill]).reshape(_EPAD // _CH, _CH)
    ew2d = jnp.concatenate(
        [edge_attr, jnp.zeros((pad,), jnp.float32)]).reshape(_EPAD // _CH, _CH)

    zeros1 = jnp.zeros((_N,), jnp.float32)
    zeros128 = jnp.zeros((_ZR, 128), jnp.float32)

    d0, d1 = _sc_deg(dst2d, ew2d, zeros1)
    h1, u1, dv, dv2 = _tc_prep(jnp.stack((d0, d1), axis=1), x, W1)
    S1 = _prop128(src2d, dst2d, ew2d, u1, zeros128)
    y2, u2 = _tc_l1(S1[0], S1[1], h1, dv, dv2, b1.reshape(1, 64), W2)
    S2 = _prop128(src2d, dst2d, ew2d, u2, zeros128)
    z2, u3 = _tc_l2(S2[0], S2[1], y2, dv, dv2, b2.reshape(1, 128))
    S3 = _prop128(src2d, dst2d, ew2d, u3, zeros128)
    return _tc_final(S3[0], S3[1], z2, dv, dv2, W3, b3.reshape(1, 256),
                     Wl1, bl1.reshape(1, 128), Wl2, bl2.reshape(1, 10))


# speedup vs baseline: 16.1590x; 1.0003x over previous
import functools

import jax
import jax.numpy as jnp
from jax import lax
from jax.experimental import pallas as pl
from jax.experimental.pallas import tpu as pltpu
from jax.experimental.pallas import tpu_sc as plsc

_N = 10000
_E = 320000
_CH = 128
_CPT = 80
_EPAD = 32 * _CPT * _CH
_ZR = 2000

_mesh = plsc.VectorSubcoreMesh(core_axis_name="c", subcore_axis_name="s",
                               num_cores=2, num_subcores=16)


@functools.partial(
    pl.kernel,
    out_type=[jax.ShapeDtypeStruct((_N,), jnp.float32),
              jax.ShapeDtypeStruct((_N,), jnp.float32)],
    mesh=_mesh,
    scratch_types=[
        pltpu.VMEM((_CPT, _CH), jnp.int32),
        pltpu.VMEM((_CPT, _CH), jnp.float32),
        pltpu.VMEM_SHARED((_N,), jnp.float32),
    ],
)
def _sc_deg(dst_h, ew_h, z_h, out0_h, out1_h, dstb, ewb, acc):
    c = lax.axis_index("c")
    s = lax.axis_index("s")
    row0 = c * (16 * _CPT) + s * _CPT
    pltpu.sync_copy(dst_h.at[pl.ds(row0, _CPT), :], dstb)
    pltpu.sync_copy(ew_h.at[pl.ds(row0, _CPT), :], ewb)

    @pl.when(s == 0)
    def _zero():
        pltpu.sync_copy(z_h, acc)

    plsc.subcore_barrier()

    def chunk(i, carry):
        pltpu.sync_copy(ewb.at[i], acc.at[dstb.at[i]], add=True)
        return carry

    lax.fori_loop(0, _CPT, chunk, 0)
    plsc.subcore_barrier()

    @pl.when((s == 0) & (c == 0))
    def _drain0():
        pltpu.sync_copy(acc, out0_h)

    @pl.when((s == 0) & (c == 1))
    def _drain1():
        pltpu.sync_copy(acc, out1_h)


def _make_prop(F):
    @functools.partial(
        pl.kernel,
        out_type=jax.ShapeDtypeStruct((2, _N, F), jnp.float32),
        mesh=_mesh,
        scratch_types=[
            pltpu.VMEM((_CPT, _CH), jnp.int32),
            pltpu.VMEM((_CPT, _CH), jnp.int32),
            pltpu.VMEM((_CPT, _CH), jnp.float32),
            pltpu.VMEM((_CH, F), jnp.float32),
            pltpu.VMEM_SHARED((_N, F), jnp.float32),
            pltpu.SemaphoreType.DMA,
        ],
    )
    def prop(src_h, dst_h, ew_h, u_h, z_h, out_h, srcb, dstb, ewb, rows,
             acc, gsem):
        c = lax.axis_index("c")
        s = lax.axis_index("s")
        row0 = c * (16 * _CPT) + s * _CPT
        pltpu.sync_copy(src_h.at[pl.ds(row0, _CPT), :], srcb)
        pltpu.sync_copy(dst_h.at[pl.ds(row0, _CPT), :], dstb)
        pltpu.sync_copy(ew_h.at[pl.ds(row0, _CPT), :], ewb)

        @pl.when(s < 5)
        def _zero():
            pltpu.sync_copy(z_h, acc.at[pl.ds(s * _ZR, _ZR), :])

        plsc.subcore_barrier()

        def scale(buf, i):
            def sbody(j, c2):
                wv = ewb[i, pl.ds(j * 16, 16)]
                for l in range(16):
                    e = j * 16 + l
                    w = wv[l]
                    for f in range(F // 16):
                        sl = pl.ds(f * 16, 16)
                        buf[e, sl] = buf[e, sl] * w
                return c2

            lax.fori_loop(0, _CH // 16, sbody, 0)

        def chunk(i, carry):
            pltpu.async_copy(u_h.at[srcb.at[i]], rows, gsem).wait()
            scale(rows, i)
            pltpu.sync_copy(rows, acc.at[dstb.at[i]], add=True)
            return carry

        lax.fori_loop(0, _CPT, chunk, 0)
        plsc.subcore_barrier()

        @pl.when(s < 5)
        def _drain():
            pltpu.sync_copy(acc.at[pl.ds(s * _ZR, _ZR), :],
                            out_h.at[c, pl.ds(s * _ZR, _ZR), :])

    return prop


_prop128 = _make_prop(128)

_R = 2000


def _tc_prep(deg_t, x, W1):
    def body(deg_ref, x_ref, w_ref, h_ref, u_ref, dv_ref, dv2_ref):
        dsum = deg_ref[:, 0:1] + deg_ref[:, 1:2] + 1.0
        dv = lax.rsqrt(dsum)
        h = jnp.dot(x_ref[...], w_ref[...], preferred_element_type=jnp.float32)
        h_ref[...] = h
        u_ref[...] = jnp.concatenate([h * dv, jnp.zeros_like(h)], axis=1)
        dv_ref[...] = dv
        dv2_ref[...] = dv * dv

    return pl.pallas_call(
        body,
        grid=(_N // _R,),
        in_specs=[
            pl.BlockSpec((_R, 2), lambda r: (r, 0)),
            pl.BlockSpec((_R, 128), lambda r: (r, 0)),
            pl.BlockSpec((128, 64), lambda r: (0, 0)),
        ],
        out_specs=[
            pl.BlockSpec((_R, 64), lambda r: (r, 0)),
            pl.BlockSpec((_R, 128), lambda r: (r, 0)),
            pl.BlockSpec((_R, 1), lambda r: (r, 0)),
            pl.BlockSpec((_R, 1), lambda r: (r, 0)),
        ],
        out_shape=[
            jax.ShapeDtypeStruct((_N, 64), jnp.float32),
            jax.ShapeDtypeStruct((_N, 128), jnp.float32),
            jax.ShapeDtypeStruct((_N, 1), jnp.float32),
            jax.ShapeDtypeStruct((_N, 1), jnp.float32),
        ],
    )(deg_t, x, W1)


def _tc_l1(sa, sb, h1, dv, dv2, b1, W2):
    def body(sa_ref, sb_ref, h_ref, dv_ref, dv2_ref, b_ref, w_ref,
             y_ref, u_ref):
        t = (dv_ref[...] * (sa_ref[...] + sb_ref[...])[:, 0:64]
             + dv2_ref[...] * h_ref[...] + b_ref[...])
        z1 = jnp.maximum(t, 0.0)
        y2 = jnp.dot(z1, w_ref[...], preferred_element_type=jnp.float32)
        y_ref[...] = y2
        u_ref[...] = dv_ref[...] * y2

    return pl.pallas_call(
        body,
        grid=(_N // _R,),
        in_specs=[
            pl.BlockSpec((_R, 128), lambda r: (r, 0)),
            pl.BlockSpec((_R, 128), lambda r: (r, 0)),
            pl.BlockSpec((_R, 64), lambda r: (r, 0)),
            pl.BlockSpec((_R, 1), lambda r: (r, 0)),
            pl.BlockSpec((_R, 1), lambda r: (r, 0)),
            pl.BlockSpec((1, 64), lambda r: (0, 0)),
            pl.BlockSpec((64, 128), lambda r: (0, 0)),
        ],
        out_specs=[
            pl.BlockSpec((_R, 128), lambda r: (r, 0)),
            pl.BlockSpec((_R, 128), lambda r: (r, 0)),
        ],
        out_shape=[
            jax.ShapeDtypeStruct((_N, 128), jnp.float32),
            jax.ShapeDtypeStruct((_N, 128), jnp.float32),
        ],
    )(sa, sb, h1, dv, dv2, b1, W2)


def _tc_l2(sa, sb, y2, dv, dv2, b2):
    def body(sa_ref, sb_ref, y_ref, dv_ref, dv2_ref, b_ref, z_ref, u_ref):
        t = (dv_ref[...] * (sa_ref[...] + sb_ref[...])
             + dv2_ref[...] * y_ref[...] + b_ref[...])
        z2 = jnp.maximum(t, 0.0)
        z_ref[...] = z2
        u_ref[...] = dv_ref[...] * z2

    return pl.pallas_call(
        body,
        grid=(_N // _R,),
        in_specs=[
            pl.BlockSpec((_R, 128), lambda r: (r, 0)),
            pl.BlockSpec((_R, 128), lambda r: (r, 0)),
            pl.BlockSpec((_R, 128), lambda r: (r, 0)),
            pl.BlockSpec((_R, 1), lambda r: (r, 0)),
            pl.BlockSpec((_R, 1), lambda r: (r, 0)),
            pl.BlockSpec((1, 128), lambda r: (0, 0)),
        ],
        out_specs=[
            pl.BlockSpec((_R, 128), lambda r: (r, 0)),
            pl.BlockSpec((_R, 128), lambda r: (r, 0)),
        ],
        out_shape=[
            jax.ShapeDtypeStruct((_N, 128), jnp.float32),
            jax.ShapeDtypeStruct((_N, 128), jnp.float32),
        ],
    )(sa, sb, y2, dv, dv2, b2)


def _tc_final(sa, sb, z2, dv, dv2, W3, b3, Wl1, bl1, Wl2, bl2):
    G = _N // _R

    def body(sa_ref, sb_ref, z2_ref, dv_ref, dv2_ref, w3_ref, b3_ref,
             wl1_ref, bl1_ref, wl2_ref, bl2_ref, out_ref, gmax):
        r = pl.program_id(0)
        t = (dv_ref[...] * (sa_ref[...] + sb_ref[...])
             + dv2_ref[...] * z2_ref[...])
        h3 = jnp.maximum(
            jnp.dot(t, w3_ref[...], preferred_element_type=jnp.float32)
            + b3_ref[...], 0.0)
        m = jnp.broadcast_to(jnp.max(h3, axis=0, keepdims=True), (8, 256))

        @pl.when(r == 0)
        def _init():
            gmax[...] = m

        @pl.when(r > 0)
        def _acc():
            gmax[...] = jnp.maximum(gmax[...], m)

        @pl.when(r == G - 1)
        def _fin():
            g = gmax[0:1, :]
            a = jnp.maximum(
                jnp.dot(g, wl1_ref[...], preferred_element_type=jnp.float32)
                + bl1_ref[...], 0.0)
            logits = (jnp.dot(a, wl2_ref[...],
                              preferred_element_type=jnp.float32)
                      + bl2_ref[...])
            m0 = jnp.max(logits, axis=1, keepdims=True)
            lse = m0 + jnp.log(
                jnp.sum(jnp.exp(logits - m0), axis=1, keepdims=True))
            out_ref[...] = logits - lse

    return pl.pallas_call(
        body,
        grid=(G,),
        in_specs=[
            pl.BlockSpec((_R, 128), lambda r: (r, 0)),
            pl.BlockSpec((_R, 128), lambda r: (r, 0)),
            pl.BlockSpec((_R, 128), lambda r: (r, 0)),
            pl.BlockSpec((_R, 1), lambda r: (r, 0)),
            pl.BlockSpec((_R, 1), lambda r: (r, 0)),
            pl.BlockSpec((128, 256), lambda r: (0, 0)),
            pl.BlockSpec((1, 256), lambda r: (0, 0)),
            pl.BlockSpec((256, 128), lambda r: (0, 0)),
            pl.BlockSpec((1, 128), lambda r: (0, 0)),
            pl.BlockSpec((128, 10), lambda r: (0, 0)),
            pl.BlockSpec((1, 10), lambda r: (0, 0)),
        ],
        out_specs=pl.BlockSpec((1, 10), lambda r: (0, 0)),
        out_shape=jax.ShapeDtypeStruct((1, 10), jnp.float32),
        scratch_shapes=[pltpu.VMEM((8, 256), jnp.float32)],
    )(sa, sb, z2, dv, dv2, W3, b3, Wl1, bl1, Wl2, bl2)


def kernel(x, edge_index, edge_attr, W1, b1, W2, b2, W3, b3, Wl1, bl1, Wl2,
           bl2):
    pad = _EPAD - _E
    fill = jnp.arange(pad, dtype=jnp.int32)
    src2d = jnp.concatenate([edge_index[0], fill]).reshape(_EPAD // _CH, _CH)
    dst2d = jnp.concatenate([edge_index[1], fill]).reshape(_EPAD // _CH, _CH)
    ew2d = jnp.concatenate(
        [edge_attr, jnp.zeros((pad,), jnp.float32)]).reshape(_EPAD // _CH, _CH)

    zeros1 = jnp.zeros((_N,), jnp.float32)
    zeros128 = jnp.zeros((_ZR, 128), jnp.float32)

    d0, d1 = _sc_deg(dst2d, ew2d, zeros1)
    h1, u1, dv, dv2 = _tc_prep(jnp.stack((d0, d1), axis=1), x, W1)
    S1 = _prop128(src2d, dst2d, ew2d, u1, zeros128)
    y2, u2 = _tc_l1(S1[0], S1[1], h1, dv, dv2, b1.reshape(1, 64), W2)
    S2 = _prop128(src2d, dst2d, ew2d, u2, zeros128)
    z2, u3 = _tc_l2(S2[0], S2[1], y2, dv, dv2, b2.reshape(1, 128))
    S3 = _prop128(src2d, dst2d, ew2d, u3, zeros128)
    return _tc_final(S3[0], S3[1], z2, dv, dv2, W3, b3.reshape(1, 256),
                     Wl1, bl1.reshape(1, 128), Wl2, bl2.reshape(1, 10))


# speedup vs baseline: 16.1697x; 1.0007x over previous
import functools

import jax
import jax.numpy as jnp
from jax import lax
from jax.experimental import pallas as pl
from jax.experimental.pallas import tpu as pltpu
from jax.experimental.pallas import tpu_sc as plsc

_N = 10000
_E = 320000
_CH = 128
_CPT = 80
_EPAD = 32 * _CPT * _CH
_ZR = 2000

_mesh = plsc.VectorSubcoreMesh(core_axis_name="c", subcore_axis_name="s",
                               num_cores=2, num_subcores=16)


@functools.partial(
    pl.kernel,
    out_type=[jax.ShapeDtypeStruct((_N,), jnp.float32),
              jax.ShapeDtypeStruct((_N,), jnp.float32)],
    mesh=_mesh,
    scratch_types=[
        pltpu.VMEM((_CPT, _CH), jnp.int32),
        pltpu.VMEM((_CPT, _CH), jnp.float32),
        pltpu.VMEM_SHARED((_N,), jnp.float32),
    ],
)
def _sc_deg(dst_h, ew_h, z_h, out0_h, out1_h, dstb, ewb, acc):
    c = lax.axis_index("c")
    s = lax.axis_index("s")
    row0 = (c * 16 + s) * _CPT
    pltpu.sync_copy(dst_h.at[pl.ds(row0, _CPT), :], dstb)
    pltpu.sync_copy(ew_h.at[pl.ds(row0, _CPT), :], ewb)

    @pl.when(s == 0)
    def _zero():
        pltpu.sync_copy(z_h, acc)

    plsc.subcore_barrier()

    def chunk(i, carry):
        pltpu.sync_copy(ewb.at[i], acc.at[dstb.at[i]], add=True)
        return carry

    lax.fori_loop(0, _CPT, chunk, 0)
    plsc.subcore_barrier()

    @pl.when((s == 0) & (c == 0))
    def _drain0():
        pltpu.sync_copy(acc, out0_h)

    @pl.when((s == 0) & (c == 1))
    def _drain1():
        pltpu.sync_copy(acc, out1_h)


def _make_prop(F):
    @functools.partial(
        pl.kernel,
        out_type=jax.ShapeDtypeStruct((2, _N, F), jnp.float32),
        mesh=_mesh,
        scratch_types=[
            pltpu.VMEM((_CPT, _CH), jnp.int32),
            pltpu.VMEM((_CPT, _CH), jnp.int32),
            pltpu.VMEM((_CPT, _CH), jnp.float32),
            pltpu.VMEM((_CH, F), jnp.float32),
            pltpu.VMEM_SHARED((_N, F), jnp.float32),
            pltpu.SemaphoreType.DMA,
        ],
    )
    def prop(src_h, dst_h, ew_h, u_h, z_h, out_h, srcb, dstb, ewb, rows,
             acc, gsem):
        c = lax.axis_index("c")
        s = lax.axis_index("s")
        row0 = (c * 16 + s) * _CPT
        pltpu.sync_copy(src_h.at[pl.ds(row0, _CPT), :], srcb)
        pltpu.sync_copy(dst_h.at[pl.ds(row0, _CPT), :], dstb)
        pltpu.sync_copy(ew_h.at[pl.ds(row0, _CPT), :], ewb)

        @pl.when(s < 5)
        def _zero():
            pltpu.sync_copy(z_h, acc.at[pl.ds(s * _ZR, _ZR), :])

        plsc.subcore_barrier()

        def scale(buf, i):
            def sbody(j, c2):
                wv = ewb[i, pl.ds(j * 16, 16)]
                for l in range(16):
                    e = j * 16 + l
                    w = wv[l]
                    for f in range(F // 16):
                        sl = pl.ds(f * 16, 16)
                        buf[e, sl] = buf[e, sl] * w
                return c2

            lax.fori_loop(0, _CH // 16, sbody, 0)

        def chunk(i, carry):
            pltpu.async_copy(u_h.at[srcb.at[i]], rows, gsem).wait()
            scale(rows, i)
            pltpu.sync_copy(rows, acc.at[dstb.at[i]], add=True)
            return carry

        lax.fori_loop(0, _CPT, chunk, 0)
        plsc.subcore_barrier()

        @pl.when(s < 5)
        def _drain():
            pltpu.sync_copy(acc.at[pl.ds(s * _ZR, _ZR), :],
                            out_h.at[c, pl.ds(s * _ZR, _ZR), :])

    return prop


_prop128 = _make_prop(128)

_R = 2000


def _tc_prep(deg_t, x, W1):
    def body(deg_ref, x_ref, w_ref, h_ref, u_ref, dv_ref, dv2_ref):
        dsum = deg_ref[:, 0:1] + deg_ref[:, 1:2] + 1.0
        dv = lax.rsqrt(dsum)
        h = jnp.dot(x_ref[...], w_ref[...], preferred_element_type=jnp.float32)
        h_ref[...] = h
        u_ref[...] = jnp.concatenate([h * dv, jnp.zeros_like(h)], axis=1)
        dv_ref[...] = dv
        dv2_ref[...] = dv * dv

    return pl.pallas_call(
        body,
        grid=(_N // _R,),
        in_specs=[
            pl.BlockSpec((_R, 2), lambda r: (r, 0)),
            pl.BlockSpec((_R, 128), lambda r: (r, 0)),
            pl.BlockSpec((128, 64), lambda r: (0, 0)),
        ],
        out_specs=[
            pl.BlockSpec((_R, 64), lambda r: (r, 0)),
            pl.BlockSpec((_R, 128), lambda r: (r, 0)),
            pl.BlockSpec((_R, 1), lambda r: (r, 0)),
            pl.BlockSpec((_R, 1), lambda r: (r, 0)),
        ],
        out_shape=[
            jax.ShapeDtypeStruct((_N, 64), jnp.float32),
            jax.ShapeDtypeStruct((_N, 128), jnp.float32),
            jax.ShapeDtypeStruct((_N, 1), jnp.float32),
            jax.ShapeDtypeStruct((_N, 1), jnp.float32),
        ],
    )(deg_t, x, W1)


def _tc_l1(sa, sb, h1, dv, dv2, b1, W2):
    def body(sa_ref, sb_ref, h_ref, dv_ref, dv2_ref, b_ref, w_ref,
             y_ref, u_ref):
        t = (dv_ref[...] * (sa_ref[...] + sb_ref[...])[:, 0:64]
             + dv2_ref[...] * h_ref[...] + b_ref[...])
        z1 = jnp.maximum(t, 0.0)
        y2 = jnp.dot(z1, w_ref[...], preferred_element_type=jnp.float32)
        y_ref[...] = y2
        u_ref[...] = dv_ref[...] * y2

    return pl.pallas_call(
        body,
        grid=(_N // _R,),
        in_specs=[
            pl.BlockSpec((_R, 128), lambda r: (r, 0)),
            pl.BlockSpec((_R, 128), lambda r: (r, 0)),
            pl.BlockSpec((_R, 64), lambda r: (r, 0)),
            pl.BlockSpec((_R, 1), lambda r: (r, 0)),
            pl.BlockSpec((_R, 1), lambda r: (r, 0)),
            pl.BlockSpec((1, 64), lambda r: (0, 0)),
            pl.BlockSpec((64, 128), lambda r: (0, 0)),
        ],
        out_specs=[
            pl.BlockSpec((_R, 128), lambda r: (r, 0)),
            pl.BlockSpec((_R, 128), lambda r: (r, 0)),
        ],
        out_shape=[
            jax.ShapeDtypeStruct((_N, 128), jnp.float32),
            jax.ShapeDtypeStruct((_N, 128), jnp.float32),
        ],
    )(sa, sb, h1, dv, dv2, b1, W2)


def _tc_l2(sa, sb, y2, dv, dv2, b2):
    def body(sa_ref, sb_ref, y_ref, dv_ref, dv2_ref, b_ref, z_ref, u_ref):
        t = (dv_ref[...] * (sa_ref[...] + sb_ref[...])
             + dv2_ref[...] * y_ref[...] + b_ref[...])
        z2 = jnp.maximum(t, 0.0)
        z_ref[...] = z2
        u_ref[...] = dv_ref[...] * z2

    return pl.pallas_call(
        body,
        grid=(_N // _R,),
        in_specs=[
            pl.BlockSpec((_R, 128), lambda r: (r, 0)),
            pl.BlockSpec((_R, 128), lambda r: (r, 0)),
            pl.BlockSpec((_R, 128), lambda r: (r, 0)),
            pl.BlockSpec((_R, 1), lambda r: (r, 0)),
            pl.BlockSpec((_R, 1), lambda r: (r, 0)),
            pl.BlockSpec((1, 128), lambda r: (0, 0)),
        ],
        out_specs=[
            pl.BlockSpec((_R, 128), lambda r: (r, 0)),
            pl.BlockSpec((_R, 128), lambda r: (r, 0)),
        ],
        out_shape=[
            jax.ShapeDtypeStruct((_N, 128), jnp.float32),
            jax.ShapeDtypeStruct((_N, 128), jnp.float32),
        ],
    )(sa, sb, y2, dv, dv2, b2)


def _tc_final(sa, sb, z2, dv, dv2, W3, b3, Wl1, bl1, Wl2, bl2):
    G = _N // _R

    def body(sa_ref, sb_ref, z2_ref, dv_ref, dv2_ref, w3_ref, b3_ref,
             wl1_ref, bl1_ref, wl2_ref, bl2_ref, out_ref, gmax):
        r = pl.program_id(0)
        t = (dv_ref[...] * (sa_ref[...] + sb_ref[...])
             + dv2_ref[...] * z2_ref[...])
        h3 = jnp.maximum(
            jnp.dot(t, w3_ref[...], preferred_element_type=jnp.float32)
            + b3_ref[...], 0.0)
        m = jnp.broadcast_to(jnp.max(h3, axis=0, keepdims=True), (8, 256))

        @pl.when(r == 0)
        def _init():
            gmax[...] = m

        @pl.when(r > 0)
        def _acc():
            gmax[...] = jnp.maximum(gmax[...], m)

        @pl.when(r == G - 1)
        def _fin():
            g = gmax[0:1, :]
            a = jnp.maximum(
                jnp.dot(g, wl1_ref[...], preferred_element_type=jnp.float32)
                + bl1_ref[...], 0.0)
            logits = (jnp.dot(a, wl2_ref[...],
                              preferred_element_type=jnp.float32)
                      + bl2_ref[...])
            m0 = jnp.max(logits, axis=1, keepdims=True)
            lse = m0 + jnp.log(
                jnp.sum(jnp.exp(logits - m0), axis=1, keepdims=True))
            out_ref[...] = logits - lse

    return pl.pallas_call(
        body,
        grid=(G,),
        in_specs=[
            pl.BlockSpec((_R, 128), lambda r: (r, 0)),
            pl.BlockSpec((_R, 128), lambda r: (r, 0)),
            pl.BlockSpec((_R, 128), lambda r: (r, 0)),
            pl.BlockSpec((_R, 1), lambda r: (r, 0)),
            pl.BlockSpec((_R, 1), lambda r: (r, 0)),
            pl.BlockSpec((128, 256), lambda r: (0, 0)),
            pl.BlockSpec((1, 256), lambda r: (0, 0)),
            pl.BlockSpec((256, 128), lambda r: (0, 0)),
            pl.BlockSpec((1, 128), lambda r: (0, 0)),
            pl.BlockSpec((128, 10), lambda r: (0, 0)),
            pl.BlockSpec((1, 10), lambda r: (0, 0)),
        ],
        out_specs=pl.BlockSpec((1, 10), lambda r: (0, 0)),
        out_shape=jax.ShapeDtypeStruct((1, 10), jnp.float32),
        scratch_shapes=[pltpu.VMEM((8, 256), jnp.float32)],
    )(sa, sb, z2, dv, dv2, W3, b3, Wl1, bl1, Wl2, bl2)


def kernel(x, edge_index, edge_attr, W1, b1, W2, b2, W3, b3, Wl1, bl1, Wl2,
           bl2):
    pad = _EPAD - _E
    fill = jnp.arange(pad, dtype=jnp.int32) % _N
    src1 = jnp.concatenate([edge_index[0], fill])
    dst1 = jnp.concatenate([edge_index[1], fill])
    ew1 = jnp.concatenate([edge_attr, jnp.zeros((pad,), jnp.float32)])
    shp = (_EPAD // _CH, _CH)
    src2d = src1.reshape(shp)
    dst2d = dst1.reshape(shp)
    ew2d = ew1.reshape(shp)

    zeros1 = jnp.zeros((_N,), jnp.float32)
    zeros128 = jnp.zeros((_ZR, 128), jnp.float32)

    d0, d1 = _sc_deg(dst2d, ew2d, zeros1)
    h1, u1, dv, dv2 = _tc_prep(jnp.stack((d0, d1), axis=1), x, W1)
    S1 = _prop128(src2d, dst2d, ew2d, u1, zeros128)
    y2, u2 = _tc_l1(S1[0], S1[1], h1, dv, dv2, b1.reshape(1, 64), W2)
    S2 = _prop128(src2d, dst2d, ew2d, u2, zeros128)
    z2, u3 = _tc_l2(S2[0], S2[1], y2, dv, dv2, b2.reshape(1, 128))
    S3 = _prop128(src2d, dst2d, ew2d, u3, zeros128)
    return _tc_final(S3[0], S3[1], z2, dv, dv2, W3, b3.reshape(1, 256),
                     Wl1, bl1.reshape(1, 128), Wl2, bl2.reshape(1, 10))
